# baseline (speedup 1.0000x reference)
import jax
import jax.numpy as jnp
from jax import lax
from jax.experimental import pallas as pl
from jax.experimental.pallas import tpu as pltpu
from jax.experimental.pallas import tpu_sc as plsc

N = 1000
NPAD = 1024
B = 16384
NC = 2
NS = 16
NW = NC * NS
BPW = B // NW
LANES = 16
CHUNK = 128
NCHUNK = BPW // CHUNK


def _fold_body(wc_hbm, in_hbm, wa_ref, wa8_ref, t2_ref, ab_ref,
               wc_v, in_v, sem):
    cp_w = pltpu.async_copy(wc_hbm, wc_v, sem)
    cp_i = pltpu.async_copy(in_hbm, in_v, sem)

    cp_i.wait()
    xi = jax.lax.bitcast_convert_type(in_v[...], jnp.float32)
    ti = jnp.transpose(xi, (1, 0))
    ab_ref[...] = jax.lax.bitcast_convert_type(ti, jnp.int32)

    cp_w.wait()
    x = wc_v[...]
    m = jnp.max(x, axis=1, keepdims=True)
    s = jnp.sum(jnp.exp(x - m), axis=1, keepdims=True)
    lse_c = m + jnp.log(s)
    wa8 = wa8_ref[...]
    ma = jnp.max(wa8)
    sa = jnp.sum(jnp.exp(wa8 - ma))
    lse_a = ma + jnp.log(sa)
    t2 = x + (wa_ref[...] - lse_a - lse_c)
    t2p = jnp.concatenate(
        [t2, jnp.zeros((N, NPAD - N), jnp.float32)], axis=1)
    for rg in range(N // 8):
        for tj in range(NPAD // 128):
            t2_ref[pl.ds(rg * 64 + tj * 8, 8), :] = (
                t2p[rg * 8:(rg + 1) * 8, tj * 128:(tj + 1) * 128])


def _sc_body(ab_hbm, t2_hbm, out_hbm, a_v, b_v, idx_v, out_v, sem, gsem):
    wid = lax.axis_index("s") * NC + lax.axis_index("c")
    row0 = wid * NCHUNK

    cp_a = pltpu.async_copy(ab_hbm.at[0, pl.ds(row0, NCHUNK)], a_v, sem)
    cp_b = pltpu.async_copy(ab_hbm.at[1, pl.ds(row0, NCHUNK)], b_v, sem)
    cp_a.wait()
    cp_b.wait()

    for j in range(NCHUNK):
        for k in range(CHUNK // LANES):
            sl = pl.ds(k * LANES, LANES)
            a = a_v[j, sl]
            b = b_v[j, sl]
            idx_v[j, sl] = ((a >> 3) * 8192 + (a & 7) * 128
                            + (b >> 7) * 1024 + (b & 127))

    gathers = [
        pltpu.async_copy(t2_hbm.at[idx_v.at[j]], out_v.at[j], gsem)
        for j in range(NCHUNK)
    ]
    for cp in gathers:
        cp.wait()

    pltpu.sync_copy(out_v, out_hbm.at[pl.ds(row0, NCHUNK)])


@jax.jit
def kernel(inputs, w_A, w_cond):
    inputs = inputs.astype(jnp.int32)
    w_A = w_A.astype(jnp.float32)
    w_cond = w_cond.astype(jnp.float32)

    table2, ab = pl.pallas_call(
        _fold_body,
        in_specs=[
            pl.BlockSpec(memory_space=pl.ANY),
            pl.BlockSpec(memory_space=pl.ANY),
            pl.BlockSpec((N, 1), lambda: (0, 0)),
            pl.BlockSpec((8, 125), lambda: (0, 0)),
        ],
        out_specs=[
            pl.BlockSpec((N * NPAD // 128, 128), lambda: (0, 0)),
            pl.BlockSpec((2, B), lambda: (0, 0)),
        ],
        out_shape=[
            jax.ShapeDtypeStruct((N * NPAD // 128, 128), jnp.float32),
            jax.ShapeDtypeStruct((2, B), jnp.int32),
        ],
        scratch_shapes=[
            pltpu.VMEM((N, N), jnp.float32),
            pltpu.VMEM((B, 2), jnp.int32),
            pltpu.SemaphoreType.DMA,
        ],
    )(w_cond, inputs, w_A[:, None], w_A.reshape(8, 125))

    t2_flat = table2.reshape(N * NPAD)
    ab3 = ab.reshape(2, B // CHUNK, CHUNK)

    sc_kernel = pl.kernel(
        _sc_body,
        out_type=jax.ShapeDtypeStruct((B // CHUNK, CHUNK), jnp.float32),
        mesh=plsc.VectorSubcoreMesh(core_axis_name="c", subcore_axis_name="s"),
        scratch_types=[
            pltpu.VMEM((NCHUNK, CHUNK), jnp.int32),
            pltpu.VMEM((NCHUNK, CHUNK), jnp.int32),
            pltpu.VMEM((NCHUNK, CHUNK), jnp.int32),
            pltpu.VMEM((NCHUNK, CHUNK), jnp.float32),
            pltpu.SemaphoreType.DMA,
            pltpu.SemaphoreType.DMA,
        ],
    )
    out2 = sc_kernel(ab3, t2_flat)
    return out2.reshape(B)
